# Initial kernel scaffold; baseline (speedup 1.0000x reference)
import jax
import jax.numpy as jnp
from jax.experimental import pallas as pl


def kernel(tokens, oov_features, fixed_weights):
    raise NotImplementedError("write your pallas kernel here")


# speedup vs baseline: 6.0057x; 6.0057x over previous
import functools

import jax
import jax.numpy as jnp
from jax import lax
from jax.experimental import pallas as pl
from jax.experimental.pallas import tpu as pltpu
from jax.experimental.pallas import tpu_sc as plsc

_BS = 64
_SEQ = 200
_VOCAB = 1000
_N_OOV = 50
_D = 1024
_PAD = 0

_N_TOKENS = _BS * _SEQ
_TABLE_ROWS = _VOCAB + _BS * _N_OOV
_LANES = 16
_CHUNK = 80


@functools.cache
def _build_gather():
    info = plsc.get_sparse_core_info()
    nc, ns = info.num_cores, info.num_subcores
    nw = nc * ns
    per_w = _N_TOKENS // nw
    assert per_w % _CHUNK == 0 and per_w % _LANES == 0
    n_chunks = per_w // _CHUNK
    n_vec = per_w // _LANES
    mesh = plsc.VectorSubcoreMesh(
        core_axis_name="c", subcore_axis_name="s")

    @functools.partial(
        pl.kernel,
        out_type=jax.ShapeDtypeStruct((_N_TOKENS, _D), jnp.float32),
        mesh=mesh,
        scratch_types=[
            pltpu.VMEM((per_w,), jnp.int32),
            pltpu.VMEM((_CHUNK, _D), jnp.float32),
            pltpu.SemaphoreType.DMA,
        ],
    )
    def gather_kernel(table_hbm, tokens_hbm, out_hbm, idx_v, rows_v, sem):
        wid = lax.axis_index("s") * nc + lax.axis_index("c")
        base = wid * per_w
        pltpu.sync_copy(tokens_hbm.at[pl.ds(base, per_w)], idx_v)
        lanes = lax.iota(jnp.int32, _LANES)
        woff = jnp.full((_LANES,), wid * (2 * _N_OOV), dtype=jnp.int32)
        for i in range(n_vec):
            t = idx_v[pl.ds(i * _LANES, _LANES)]
            in_b1 = (i * _LANES + lanes) >= _SEQ
            off = woff + jnp.where(in_b1, _N_OOV, 0)
            idx_v[pl.ds(i * _LANES, _LANES)] = jnp.where(
                t < _VOCAB, t, t + off)
        for c in range(n_chunks):
            pltpu.async_copy(
                table_hbm.at[idx_v.at[pl.ds(c * _CHUNK, _CHUNK)]],
                rows_v, sem).wait()
            pltpu.sync_copy(
                rows_v, out_hbm.at[pl.ds(base + c * _CHUNK, _CHUNK)])

    return gather_kernel


def kernel(tokens, oov_features, fixed_weights):
    tokens_i32 = tokens.astype(jnp.int32)
    flat_tokens = tokens_i32.reshape(_N_TOKENS)
    table = jnp.concatenate(
        [fixed_weights, oov_features.reshape(_BS * _N_OOV, _D)], axis=0)
    features = _build_gather()(table, flat_tokens).reshape(_BS, _SEQ, _D)
    padding_mask = (tokens == _PAD)[:, None, None, :]
    sequential_mask = jnp.triu(jnp.ones((_SEQ, _SEQ), dtype=bool), k=1)
    return (features, (padding_mask, sequential_mask))


# speedup vs baseline: 6.0653x; 1.0099x over previous
import functools

import jax
import jax.numpy as jnp
from jax import lax
from jax.experimental import pallas as pl
from jax.experimental.pallas import tpu as pltpu
from jax.experimental.pallas import tpu_sc as plsc

_BS = 64
_SEQ = 200
_VOCAB = 1000
_N_OOV = 50
_D = 1024
_PAD = 0

_N_TOKENS = _BS * _SEQ
_TABLE_ROWS = _VOCAB + _BS * _N_OOV
_LANES = 16
_CHUNK = 40


@functools.cache
def _build_gather():
    info = plsc.get_sparse_core_info()
    nc, ns = info.num_cores, info.num_subcores
    nw = nc * ns
    per_w = _N_TOKENS // nw
    assert per_w % _CHUNK == 0 and per_w % _LANES == 0
    n_chunks = per_w // _CHUNK
    n_vec = per_w // _LANES
    mesh = plsc.VectorSubcoreMesh(
        core_axis_name="c", subcore_axis_name="s")

    @functools.partial(
        pl.kernel,
        out_type=jax.ShapeDtypeStruct((_N_TOKENS, _D), jnp.float32),
        mesh=mesh,
        scratch_types=[
            pltpu.VMEM((per_w,), jnp.int32),
            pltpu.VMEM((_CHUNK, _D), jnp.float32),
            pltpu.VMEM((_CHUNK, _D), jnp.float32),
            pltpu.SemaphoreType.DMA,
            pltpu.SemaphoreType.DMA,
            pltpu.SemaphoreType.DMA,
            pltpu.SemaphoreType.DMA,
        ],
    )
    def gather_kernel(table_hbm, tokens_hbm, out_hbm, idx_v,
                      rows_a, rows_b, g_sem_a, g_sem_b, o_sem_a, o_sem_b):
        wid = lax.axis_index("s") * nc + lax.axis_index("c")
        base = wid * per_w
        pltpu.sync_copy(tokens_hbm.at[pl.ds(base, per_w)], idx_v)
        lanes = lax.iota(jnp.int32, _LANES)
        woff = jnp.full((_LANES,), wid * (2 * _N_OOV), dtype=jnp.int32)
        for i in range(n_vec):
            t = idx_v[pl.ds(i * _LANES, _LANES)]
            in_b1 = (i * _LANES + lanes) >= _SEQ
            off = woff + jnp.where(in_b1, _N_OOV, 0)
            idx_v[pl.ds(i * _LANES, _LANES)] = jnp.where(
                t < _VOCAB, t, t + off)
        bufs = (rows_a, rows_b)
        g_sems = (g_sem_a, g_sem_b)
        o_sems = (o_sem_a, o_sem_b)

        def start_gather(c):
            return pltpu.async_copy(
                table_hbm.at[idx_v.at[pl.ds(c * _CHUNK, _CHUNK)]],
                bufs[c % 2], g_sems[c % 2])

        def start_out(c):
            return pltpu.async_copy(
                bufs[c % 2], out_hbm.at[pl.ds(base + c * _CHUNK, _CHUNK)],
                o_sems[c % 2])

        gather_d = [None] * n_chunks
        out_d = [None] * n_chunks
        for c in range(n_chunks):
            if c >= 2:
                out_d[c - 2].wait()
            gather_d[c] = start_gather(c)
            if c >= 1:
                gather_d[c - 1].wait()
                out_d[c - 1] = start_out(c - 1)
        gather_d[n_chunks - 1].wait()
        out_d[n_chunks - 1] = start_out(n_chunks - 1)
        if n_chunks >= 2:
            out_d[n_chunks - 2].wait()
        out_d[n_chunks - 1].wait()

    return gather_kernel


def kernel(tokens, oov_features, fixed_weights):
    tokens_i32 = tokens.astype(jnp.int32)
    flat_tokens = tokens_i32.reshape(_N_TOKENS)
    table = jnp.concatenate(
        [fixed_weights, oov_features.reshape(_BS * _N_OOV, _D)], axis=0)
    features = _build_gather()(table, flat_tokens).reshape(_BS, _SEQ, _D)
    padding_mask = (tokens == _PAD)[:, None, None, :]
    sequential_mask = jnp.triu(jnp.ones((_SEQ, _SEQ), dtype=bool), k=1)
    return (features, (padding_mask, sequential_mask))
